# baseline (speedup 1.0000x reference)
import functools

import jax
import jax.numpy as jnp
from jax import lax
from jax.experimental import pallas as pl
from jax.experimental.pallas import tpu as pltpu
from jax.experimental.pallas import tpu_sc as plsc

NC = 2
NS = 16
NW = NC * NS
CH = 32
LANES = 16


def _make_sc_call(b, l, d):
  n_rows = b * l
  cpb = l // CH
  fa_per_w = (n_rows // CH) // NW
  bat_grps = NW // cpb
  b_per_w = b // bat_grps
  vregs = d // LANES
  mesh = plsc.VectorSubcoreMesh(
      core_axis_name="c", subcore_axis_name="s",
      num_cores=NC, num_subcores=NS)

  @functools.partial(
      pl.kernel,
      out_type=(jax.ShapeDtypeStruct((n_rows, d), jnp.float32),) * 3,
      mesh=mesh,
      scratch_types=[
          pltpu.VMEM((CH, d), jnp.float32),
          pltpu.VMEM((CH, d), jnp.float32),
          pltpu.VMEM((CH, d), jnp.float32),
          pltpu.VMEM((CH, d), jnp.float32),
          pltpu.VMEM((fa_per_w * CH,), jnp.int32),
          pltpu.SemaphoreType.DMA,
          pltpu.SemaphoreType.DMA,
          pltpu.SemaphoreType.DMA,
          pltpu.SemaphoreType.DMA,
          pltpu.SemaphoreType.DMA,
          pltpu.SemaphoreType.DMA,
      ],
  )
  def run(lang_h, frames_h, actions_h, pos_fa_h, emb_h,
          out_l, out_f, out_a,
          ebuf0, ebuf1, dbuf0, dbuf1, idxs, g0, g1, i0, i1, o0, o1):
    wid = lax.axis_index("s") * NC + lax.axis_index("c")
    t0 = wid * fa_per_w
    ebufs = (ebuf0, ebuf1)
    dbufs = (dbuf0, dbuf1)
    gsems = (g0, g1)
    isems = (i0, i1)
    osems = (o0, o1)

    def add_into(dst, src):
      @plsc.parallel_loop(0, CH, unroll=2)
      def _(r):
        for k in range(vregs):
          sl = pl.ds(k * LANES, LANES)
          plsc.addupdate(dst.at[r, sl], src[r, sl])

    def issue_gather(t_rel, s):
      pltpu.async_copy(emb_h.at[idxs.at[pl.ds(t_rel * CH, CH)]],
                       ebufs[s], gsems[s])

    def wait_gather(s):
      pltpu.make_async_copy(emb_h.at[idxs.at[pl.ds(0, CH)]],
                            ebufs[s], gsems[s]).wait()

    def issue_in(data_h, row0, s):
      pltpu.async_copy(data_h.at[pl.ds(row0, CH)], dbufs[s], isems[s])

    def wait_in(s):
      pltpu.make_async_copy(lang_h.at[pl.ds(0, CH)], dbufs[s],
                            isems[s]).wait()

    def issue_out(out_h, row0, s):
      pltpu.async_copy(dbufs[s], out_h.at[pl.ds(row0, CH)], osems[s])

    def wait_out(s):
      pltpu.make_async_copy(dbufs[s], out_l.at[pl.ds(0, CH)],
                            osems[s]).wait()

    def fa_row0(t_rel):
      return (t0 + t_rel) * CH

    pltpu.sync_copy(pos_fa_h.at[pl.ds(t0 * CH, fa_per_w * CH)], idxs)
    issue_gather(0, 0)
    issue_in(frames_h, fa_row0(0), 0)
    issue_in(actions_h, fa_row0(0), 1)

    def fa_body(t_rel, es, prefetch):
      if prefetch:
        issue_gather(t_rel + 1, 1 - es)
      wait_gather(es)
      wait_in(0)
      add_into(dbufs[0], ebufs[es])
      issue_out(out_f, fa_row0(t_rel), 0)
      wait_in(1)
      add_into(dbufs[1], ebufs[es])
      issue_out(out_a, fa_row0(t_rel), 1)
      wait_out(0)
      if prefetch:
        issue_in(frames_h, fa_row0(t_rel + 1), 0)
      wait_out(1)
      if prefetch:
        issue_in(actions_h, fa_row0(t_rel + 1), 1)

    @pl.loop(0, fa_per_w - 2, step=2)
    def _(t):
      fa_body(t, 0, True)
      fa_body(t + 1, 1, True)

    fa_body(fa_per_w - 2, 0, True)
    fa_body(fa_per_w - 1, 1, False)

    c = wid % cpb
    bg = wid // cpb
    base_b = bg * b_per_w
    pltpu.sync_copy(emb_h.at[pl.ds(c * CH, CH)], ebufs[0])

    def l_row0(j):
      return (base_b + j) * l + c * CH

    issue_in(lang_h, l_row0(0), 0)
    issue_in(lang_h, l_row0(1), 1)

    def l_body(j, s, prefetch):
      wait_in(s)
      add_into(dbufs[s], ebufs[0])
      issue_out(out_l, l_row0(j), s)
      wait_out(s)
      if prefetch:
        issue_in(lang_h, l_row0(j + 2), s)

    @pl.loop(0, b_per_w - 2, step=2)
    def _(j):
      l_body(j, 0, True)
      l_body(j + 1, 1, True)

    l_body(b_per_w - 2, 0, False)
    l_body(b_per_w - 1, 1, False)

  return run


def kernel(lang, frames, actions, lens_lang, lens_frames, emb):
  b, l, d = lang.shape
  n_rows = b * l

  pos_fa = (jnp.arange(l, dtype=jnp.int32)[None, :]
            + lens_lang[:, None].astype(jnp.int32)).reshape(-1)

  run = _make_sc_call(b, l, d)
  out_l, out_f, out_a = run(
      lang.reshape(n_rows, d), frames.reshape(n_rows, d),
      actions.reshape(n_rows, d), pos_fa, emb)
  return (out_l.reshape(b, l, d),
          out_f.reshape(b, l, d),
          out_a.reshape(b, l, d))


# speedup vs baseline: 1.1620x; 1.1620x over previous
import functools

import jax
import jax.numpy as jnp
from jax import lax
from jax.experimental import pallas as pl
from jax.experimental.pallas import tpu as pltpu
from jax.experimental.pallas import tpu_sc as plsc

NC = 2
NS = 16
NW = NC * NS
CH = 32
LANES = 16


def _make_sc_call(b, l, d):
  n_rows = b * l
  cpb = l // CH
  fa_per_w = (n_rows // CH) // NW
  bat_grps = NW // cpb
  b_per_w = b // bat_grps
  vregs = d // LANES
  mesh = plsc.VectorSubcoreMesh(
      core_axis_name="c", subcore_axis_name="s",
      num_cores=NC, num_subcores=NS)

  @functools.partial(
      pl.kernel,
      out_type=(jax.ShapeDtypeStruct((n_rows, d), jnp.float32),) * 2,
      mesh=mesh,
      scratch_types=[
          pltpu.VMEM((CH, d), jnp.float32),
          pltpu.VMEM((CH, d), jnp.float32),
          pltpu.VMEM((CH, d), jnp.float32),
          pltpu.VMEM((CH, d), jnp.float32),
          pltpu.VMEM((fa_per_w * CH,), jnp.int32),
          pltpu.SemaphoreType.DMA,
          pltpu.SemaphoreType.DMA,
          pltpu.SemaphoreType.DMA,
          pltpu.SemaphoreType.DMA,
          pltpu.SemaphoreType.DMA,
          pltpu.SemaphoreType.DMA,
      ],
  )
  def run(frames_h, actions_h, pos_fa_h, emb_h,
          out_f, out_a,
          ebuf0, ebuf1, dbuf0, dbuf1, idxs, g0, g1, i0, i1, o0, o1):
    wid = lax.axis_index("s") * NC + lax.axis_index("c")
    t0 = wid * fa_per_w
    ebufs = (ebuf0, ebuf1)
    dbufs = (dbuf0, dbuf1)
    gsems = (g0, g1)
    isems = (i0, i1)
    osems = (o0, o1)

    def add_into(dst, src):
      @plsc.parallel_loop(0, CH, unroll=2)
      def _(r):
        for k in range(vregs):
          sl = pl.ds(k * LANES, LANES)
          plsc.addupdate(dst.at[r, sl], src[r, sl])

    def issue_gather(t_rel, s):
      pltpu.async_copy(emb_h.at[idxs.at[pl.ds(t_rel * CH, CH)]],
                       ebufs[s], gsems[s])

    def wait_gather(s):
      pltpu.make_async_copy(emb_h.at[idxs.at[pl.ds(0, CH)]],
                            ebufs[s], gsems[s]).wait()

    def issue_in(data_h, row0, s):
      pltpu.async_copy(data_h.at[pl.ds(row0, CH)], dbufs[s], isems[s])

    def wait_in(s):
      pltpu.make_async_copy(frames_h.at[pl.ds(0, CH)], dbufs[s],
                            isems[s]).wait()

    def issue_out(out_h, row0, s):
      pltpu.async_copy(dbufs[s], out_h.at[pl.ds(row0, CH)], osems[s])

    def wait_out(s):
      pltpu.make_async_copy(dbufs[s], out_f.at[pl.ds(0, CH)],
                            osems[s]).wait()

    def fa_row0(t_rel):
      return (t0 + t_rel) * CH

    pltpu.sync_copy(pos_fa_h.at[pl.ds(t0 * CH, fa_per_w * CH)], idxs)
    issue_gather(0, 0)
    issue_in(frames_h, fa_row0(0), 0)
    issue_in(actions_h, fa_row0(0), 1)

    def fa_body(t_rel, es, prefetch):
      if prefetch:
        issue_gather(t_rel + 1, 1 - es)
      wait_gather(es)
      wait_in(0)
      add_into(dbufs[0], ebufs[es])
      issue_out(out_f, fa_row0(t_rel), 0)
      wait_in(1)
      add_into(dbufs[1], ebufs[es])
      issue_out(out_a, fa_row0(t_rel), 1)
      wait_out(0)
      if prefetch:
        issue_in(frames_h, fa_row0(t_rel + 1), 0)
      wait_out(1)
      if prefetch:
        issue_in(actions_h, fa_row0(t_rel + 1), 1)

    @pl.loop(0, fa_per_w - 2, step=2)
    def _(t):
      fa_body(t, 0, True)
      fa_body(t + 1, 1, True)

    fa_body(fa_per_w - 2, 0, True)
    fa_body(fa_per_w - 1, 1, False)

  return run


def _tc_lang_call(b, l, d):
  def body(lang_ref, emb_ref, out_ref):
    out_ref[...] = lang_ref[...] + emb_ref[...][None]

  return pl.pallas_call(
      body,
      out_shape=jax.ShapeDtypeStruct((b, l, d), jnp.float32),
      grid=(b,),
      in_specs=[
          pl.BlockSpec((1, l, d), lambda i: (i, 0, 0)),
          pl.BlockSpec((l, d), lambda i: (0, 0)),
      ],
      out_specs=pl.BlockSpec((1, l, d), lambda i: (i, 0, 0)),
  )


def kernel(lang, frames, actions, lens_lang, lens_frames, emb):
  b, l, d = lang.shape
  n_rows = b * l

  pos_fa = (jnp.arange(l, dtype=jnp.int32)[None, :]
            + lens_lang[:, None].astype(jnp.int32)).reshape(-1)

  run = _make_sc_call(b, l, d)
  out_f, out_a = run(
      frames.reshape(n_rows, d), actions.reshape(n_rows, d), pos_fa, emb)
  out_l = _tc_lang_call(b, l, d)(lang, lax.slice(emb, (0, 0), (l, d)))
  return (out_l,
          out_f.reshape(b, l, d),
          out_a.reshape(b, l, d))


# speedup vs baseline: 1.2348x; 1.0627x over previous
import functools

import jax
import jax.numpy as jnp
from jax import lax
from jax.experimental import pallas as pl
from jax.experimental.pallas import tpu as pltpu
from jax.experimental.pallas import tpu_sc as plsc

NC = 2
NS = 16
NW = NC * NS
CH = 32
LANES = 16


def _make_sc_call(n_rows, d):
  per_w = (n_rows // CH) // NW
  vregs = d // LANES
  mesh = plsc.VectorSubcoreMesh(
      core_axis_name="c", subcore_axis_name="s",
      num_cores=NC, num_subcores=NS)

  @functools.partial(
      pl.kernel,
      out_type=jax.ShapeDtypeStruct((n_rows, d), jnp.float32),
      mesh=mesh,
      scratch_types=[
          pltpu.VMEM((CH, d), jnp.float32),
          pltpu.VMEM((CH, d), jnp.float32),
          pltpu.VMEM((CH, d), jnp.float32),
          pltpu.VMEM((CH, d), jnp.float32),
          pltpu.VMEM((per_w * CH,), jnp.int32),
          pltpu.SemaphoreType.DMA,
          pltpu.SemaphoreType.DMA,
          pltpu.SemaphoreType.DMA,
          pltpu.SemaphoreType.DMA,
          pltpu.SemaphoreType.DMA,
          pltpu.SemaphoreType.DMA,
      ],
  )
  def run(data_h, pos_h, emb_h, out_h,
          ebuf0, ebuf1, dbuf0, dbuf1, idxs, g0, g1, i0, i1, o0, o1):
    wid = lax.axis_index("s") * NC + lax.axis_index("c")
    t0 = wid * per_w
    ebufs = (ebuf0, ebuf1)
    dbufs = (dbuf0, dbuf1)
    gsems = (g0, g1)
    isems = (i0, i1)
    osems = (o0, o1)

    def row0(t_rel):
      return (t0 + t_rel) * CH

    def add_into(dst, src):
      @pl.loop(0, CH)
      def _(r):
        for k in range(vregs):
          sl = pl.ds(k * LANES, LANES)
          plsc.addupdate(dst.at[r, sl], src[r, sl])

    def issue_gather(t_rel, s):
      pltpu.async_copy(emb_h.at[idxs.at[pl.ds(t_rel * CH, CH)]],
                       ebufs[s], gsems[s])

    def wait_gather(s):
      pltpu.make_async_copy(emb_h.at[idxs.at[pl.ds(0, CH)]],
                            ebufs[s], gsems[s]).wait()

    def issue_in(t_rel, s):
      pltpu.async_copy(data_h.at[pl.ds(row0(t_rel), CH)], dbufs[s],
                       isems[s])

    def wait_in(s):
      pltpu.make_async_copy(data_h.at[pl.ds(0, CH)], dbufs[s],
                            isems[s]).wait()

    def issue_out(t_rel, s):
      pltpu.async_copy(dbufs[s], out_h.at[pl.ds(row0(t_rel), CH)],
                       osems[s])

    def wait_out(s):
      pltpu.make_async_copy(dbufs[s], out_h.at[pl.ds(0, CH)],
                            osems[s]).wait()

    pltpu.sync_copy(pos_h.at[pl.ds(t0 * CH, per_w * CH)], idxs)
    issue_gather(0, 0)
    issue_in(0, 0)

    def body(t_rel, s, first, last):
      if not last:
        issue_gather(t_rel + 1, 1 - s)
      wait_gather(s)
      wait_in(s)
      add_into(dbufs[s], ebufs[s])
      issue_out(t_rel, s)
      if not first:
        wait_out(1 - s)
      if not last:
        issue_in(t_rel + 1, 1 - s)

    body(0, 0, True, False)

    @pl.loop(1, per_w - 1, step=2)
    def _(t):
      body(t, 1, False, False)
      body(t + 1, 0, False, False)

    body(per_w - 1, 1, False, True)
    wait_out(1)

  return run


def _tc_lang_call(b, l, d):
  def body(lang_ref, emb_ref, out_ref):
    out_ref[...] = lang_ref[...] + emb_ref[...][None]

  return pl.pallas_call(
      body,
      out_shape=jax.ShapeDtypeStruct((b, l, d), jnp.float32),
      grid=(b,),
      in_specs=[
          pl.BlockSpec((1, l, d), lambda i: (i, 0, 0)),
          pl.BlockSpec((l, d), lambda i: (0, 0)),
      ],
      out_specs=pl.BlockSpec((1, l, d), lambda i: (i, 0, 0)),
  )


def _tc_frames_call(b, l, d, pad_pos):
  def body(lens_ref, f_ref, emb_ref, out_ref):
    i = pl.program_id(0)
    off = lens_ref[i]
    base = pl.multiple_of((off // 8) * 8, 8)
    r = off - base
    sl = emb_ref[pl.ds(base, l + 8), :]
    rolled = pltpu.roll(sl, jnp.where(r == 0, 0, l + 8 - r), 0)
    out_ref[...] = f_ref[...] + rolled[:l, :][None]

  return pl.pallas_call(
      body,
      out_shape=jax.ShapeDtypeStruct((b, l, d), jnp.float32),
      grid=(b,),
      in_specs=[
          pl.BlockSpec(memory_space=pltpu.SMEM),
          pl.BlockSpec((1, l, d), lambda i: (i, 0, 0)),
          pl.BlockSpec((pad_pos, d), lambda i: (0, 0)),
      ],
      out_specs=pl.BlockSpec((1, l, d), lambda i: (i, 0, 0)),
  )


def kernel(lang, frames, actions, lens_lang, lens_frames, emb):
  b, l, d = lang.shape
  n_rows = b * l
  lens32 = lens_lang.astype(jnp.int32)

  pos_a = (jnp.arange(l, dtype=jnp.int32)[None, :]
           + lens32[:, None]).reshape(-1)

  pad_pos = ((l - 1) // 8 + 1) * 8 + l + 8
  emb_pad = jnp.pad(emb, ((0, max(0, pad_pos - emb.shape[0])), (0, 0)))

  out_a = _make_sc_call(n_rows, d)(actions.reshape(n_rows, d), pos_a, emb)
  out_l = _tc_lang_call(b, l, d)(lang, lax.slice(emb, (0, 0), (l, d)))
  out_f = _tc_frames_call(b, l, d, pad_pos)(lens32, frames, emb_pad)
  return (out_l, out_f, out_a.reshape(b, l, d))
